# baseline (speedup 1.0000x reference)
import functools

import jax
import jax.numpy as jnp
from jax import lax
from jax.experimental import pallas as pl
from jax.experimental.pallas import tpu as pltpu
from jax.experimental.pallas import tpu_sc as plsc

ALPHA = 0.8
GAMMA = 0.8

NC = 2
NS = 16
NW = NC * NS
LANES = 16

CH = 8192


def _sc_body(n, q_hbm, a_hbm, b_hbm, p_hbm, out_hbm, *scratch):
    bufs = (scratch[0:4], scratch[4:8])
    sin = scratch[8:10]
    sout = scratch[10:12]
    per_w = n // NW
    wid = lax.axis_index("s") * NC + lax.axis_index("c")
    base0 = wid * per_w
    nch = per_w // CH

    def in_copies(c, bset, sem):
        sl = pl.ds(base0 + c * CH, CH)
        q_v, a_v, b_v, p_v = bset
        srcs = (q_hbm.at[:, sl], a_hbm.at[sl], b_hbm.at[sl], p_hbm.at[sl])
        dsts = (q_v, a_v, b_v, p_v)
        return [pltpu.make_async_copy(s, d, sem) for s, d in zip(srcs, dsts)]

    def out_copies(c, bset, sem):
        sl = pl.ds(base0 + c * CH, CH)
        q_v = bset[0]
        return [pltpu.make_async_copy(q_v, out_hbm.at[:, sl], sem)]

    def compute(bset):
        q_v, a_v, b_v, p_v = bset
        lane = lax.iota(jnp.int32, LANES)

        @plsc.parallel_loop(0, CH // LANES, unroll=4)
        def body(i):
            s = pl.ds(i * LANES, LANES)
            av = a_v[s]
            bv = b_v[s]
            pv = p_v[s]
            pos = i * LANES + lane
            b2 = bv + bv
            pa = av + av + bv
            g0 = plsc.load_gather(q_v, [b2, pos])
            g1 = plsc.load_gather(q_v, [b2 + 1, pos])
            old = plsc.load_gather(q_v, [pa, pos])
            m = jnp.maximum(g0, g1)
            u = old + ALPHA * (pv + GAMMA * m - old)
            plsc.store_scatter(q_v, [pa, pos], u)

    for cp in in_copies(0, bufs[0], sin[0]):
        cp.start()

    def outer(t, carry):
        for b in (0, 1):
            cur = 2 * t + b
            nxt = cur + 1
            ob = 1 - b

            @pl.when(nxt < nch)
            def _prefetch():
                @pl.when(nxt >= 2)
                def _drain():
                    for cp in out_copies(nxt - 2, bufs[ob], sout[ob]):
                        cp.wait()

                for cp in in_copies(nxt, bufs[ob], sin[ob]):
                    cp.start()

            for cp in in_copies(cur, bufs[b], sin[b]):
                cp.wait()
            compute(bufs[b])
            for cp in out_copies(cur, bufs[b], sout[b]):
                cp.start()
        return carry

    lax.fori_loop(0, nch // 2, outer, 0)
    for cp in out_copies(nch - 2, bufs[0], sout[0]):
        cp.wait()
    for cp in out_copies(nch - 1, bufs[1], sout[1]):
        cp.wait()


@functools.lru_cache(maxsize=None)
def _make_sc_update(n):
    mesh = plsc.VectorSubcoreMesh(
        core_axis_name="c", subcore_axis_name="s",
        num_cores=NC, num_subcores=NS,
    )
    return pl.kernel(
        functools.partial(_sc_body, n),
        out_type=jax.ShapeDtypeStruct((4, n), jnp.float32),
        mesh=mesh,
        compiler_params=pltpu.CompilerParams(needs_layout_passes=False),
        scratch_types=(
            [pltpu.VMEM((4, CH), jnp.float32)]
            + [pltpu.VMEM((CH,), jnp.int32)] * 2
            + [pltpu.VMEM((CH,), jnp.float32)]
        ) * 2 + [pltpu.SemaphoreType.DMA] * 4,
    )


@jax.jit
def kernel(type_t_matrix, type_t1_matrix, Q_tensor, profit_matrix):
    n = type_t_matrix.size
    a = type_t_matrix.reshape(n).astype(jnp.int32)
    b = type_t1_matrix.reshape(n).astype(jnp.int32)
    p = profit_matrix.reshape(n).astype(jnp.float32)
    q4 = jnp.transpose(Q_tensor, (1, 2, 0)).reshape(4, n)

    out = _make_sc_update(n)(q4, a, b, p)
    return jnp.transpose(out.reshape(2, 2, n), (2, 0, 1))


# speedup vs baseline: 1.4244x; 1.4244x over previous
import functools

import jax
import jax.numpy as jnp
from jax import lax
from jax.experimental import pallas as pl
from jax.experimental.pallas import tpu as pltpu
from jax.experimental.pallas import tpu_sc as plsc

ALPHA = 0.8
GAMMA = 0.8

NC = 2
NS = 16
NW = NC * NS
LANES = 16

CH = 8192
SC_GRAIN = NW * CH * 2

BLK = 64 * 2048


def _sc_body(n_sc, base, q_hbm, a_hbm, b_hbm, p_hbm, out_hbm, *scratch):
    bufs = (scratch[0:4], scratch[4:8])
    sin = scratch[8:10]
    sout = scratch[10:12]
    per_w = n_sc // NW
    wid = lax.axis_index("s") * NC + lax.axis_index("c")
    base0 = wid * per_w
    nch = per_w // CH

    def in_copies(c, bset, sem):
        sl = pl.ds(base + base0 + c * CH, CH)
        q_v, a_v, b_v, p_v = bset
        srcs = (q_hbm.at[:, sl], a_hbm.at[sl], b_hbm.at[sl], p_hbm.at[sl])
        dsts = (q_v, a_v, b_v, p_v)
        return [pltpu.make_async_copy(s, d, sem) for s, d in zip(srcs, dsts)]

    def out_copies(c, bset, sem):
        sl = pl.ds(base0 + c * CH, CH)
        q_v = bset[0]
        return [pltpu.make_async_copy(q_v, out_hbm.at[:, sl], sem)]

    def compute(bset):
        q_v, a_v, b_v, p_v = bset
        lane = lax.iota(jnp.int32, LANES)

        @plsc.parallel_loop(0, CH // LANES, unroll=4)
        def body(i):
            s = pl.ds(i * LANES, LANES)
            av = a_v[s]
            bv = b_v[s]
            pv = p_v[s]
            pos = i * LANES + lane
            b2 = bv + bv
            pa = av + av + bv
            g0 = plsc.load_gather(q_v, [b2, pos])
            g1 = plsc.load_gather(q_v, [b2 + 1, pos])
            old = plsc.load_gather(q_v, [pa, pos])
            m = jnp.maximum(g0, g1)
            u = old + ALPHA * (pv + GAMMA * m - old)
            plsc.store_scatter(q_v, [pa, pos], u)

    for cp in in_copies(0, bufs[0], sin[0]):
        cp.start()

    def outer(t, carry):
        for b in (0, 1):
            cur = 2 * t + b
            nxt = cur + 1
            ob = 1 - b

            @pl.when(nxt < nch)
            def _prefetch():
                @pl.when(nxt >= 2)
                def _drain():
                    for cp in out_copies(nxt - 2, bufs[ob], sout[ob]):
                        cp.wait()

                for cp in in_copies(nxt, bufs[ob], sin[ob]):
                    cp.start()

            for cp in in_copies(cur, bufs[b], sin[b]):
                cp.wait()
            compute(bufs[b])
            for cp in out_copies(cur, bufs[b], sout[b]):
                cp.start()
        return carry

    lax.fori_loop(0, nch // 2, outer, 0)
    for cp in out_copies(nch - 2, bufs[0], sout[0]):
        cp.wait()
    for cp in out_copies(nch - 1, bufs[1], sout[1]):
        cp.wait()


@functools.lru_cache(maxsize=None)
def _make_sc_update(n_sc, base):
    mesh = plsc.VectorSubcoreMesh(
        core_axis_name="c", subcore_axis_name="s",
        num_cores=NC, num_subcores=NS,
    )
    return pl.kernel(
        functools.partial(_sc_body, n_sc, base),
        out_type=jax.ShapeDtypeStruct((4, n_sc), jnp.float32),
        mesh=mesh,
        compiler_params=pltpu.CompilerParams(needs_layout_passes=False),
        scratch_types=(
            [pltpu.VMEM((4, CH), jnp.float32)]
            + [pltpu.VMEM((CH,), jnp.int32)] * 2
            + [pltpu.VMEM((CH,), jnp.float32)]
        ) * 2 + [pltpu.SemaphoreType.DMA] * 4,
    )


def _tc_kernel(q_ref, a_ref, b_ref, p_ref, o_ref):
    q00 = q_ref[0, 0]
    q01 = q_ref[0, 1]
    q10 = q_ref[1, 0]
    q11 = q_ref[1, 1]
    a = a_ref[...]
    b = b_ref[...]
    p = p_ref[...]

    b0 = b == 0
    m = jnp.where(b0, jnp.maximum(q00, q01), jnp.maximum(q10, q11))
    old = jnp.where(
        a == 0, jnp.where(b0, q00, q01), jnp.where(b0, q10, q11)
    )
    u = old + ALPHA * (p + GAMMA * m - old)

    a0 = a == 0
    o_ref[0, 0] = jnp.where(a0 & b0, u, q00)
    o_ref[0, 1] = jnp.where(a0 & ~b0, u, q01)
    o_ref[1, 0] = jnp.where(~a0 & b0, u, q10)
    o_ref[1, 1] = jnp.where(~a0 & ~b0, u, q11)


def _tc_call(qt, a, b, p, n, n_tc):
    q_spec = pl.BlockSpec((2, 2, BLK), lambda i: (0, 0, i))
    v_spec = pl.BlockSpec((BLK,), lambda i: (i,))
    return pl.pallas_call(
        _tc_kernel,
        grid=(n_tc // BLK,),
        in_specs=[q_spec, v_spec, v_spec, v_spec],
        out_specs=q_spec,
        out_shape=jax.ShapeDtypeStruct((2, 2, n), jnp.float32),
        compiler_params=pltpu.CompilerParams(
            dimension_semantics=("arbitrary",),
        ),
    )(qt, a, b, p)


@jax.jit
def kernel(type_t_matrix, type_t1_matrix, Q_tensor, profit_matrix):
    n = type_t_matrix.size
    a = type_t_matrix.reshape(n).astype(jnp.int32)
    b = type_t1_matrix.reshape(n).astype(jnp.int32)
    p = profit_matrix.reshape(n).astype(jnp.float32)
    qt = jnp.transpose(Q_tensor, (1, 2, 0))
    q4 = qt.reshape(4, n)

    n_sc = (n // 4) // SC_GRAIN * SC_GRAIN
    if n_sc == 0 or (n - n_sc) % BLK:
        n_sc = n if n % SC_GRAIN == 0 else 0
    n_tc = n - n_sc

    if n_sc:
        sc_out = _make_sc_update(n_sc, n_tc)(q4, a, b, p)
    if n_tc:
        out = _tc_call(qt, a, b, p, n, n_tc)
        if n_sc:
            out = lax.dynamic_update_slice(
                out, sc_out.reshape(2, 2, n_sc), (0, 0, n_tc)
            )
    else:
        out = sc_out.reshape(2, 2, n)
    return jnp.transpose(out, (2, 0, 1))


# speedup vs baseline: 1.5572x; 1.0933x over previous
import functools

import jax
import jax.numpy as jnp
from jax import lax
from jax.experimental import pallas as pl
from jax.experimental.pallas import tpu as pltpu
from jax.experimental.pallas import tpu_sc as plsc

ALPHA = 0.8
GAMMA = 0.8

NC = 2
NS = 16
NW = NC * NS
LANES = 16

CH = 8192
SC_GRAIN = NW * CH * 2

BLK = 64 * 2048


def _sc_body(n_sc, base, q_hbm, a_hbm, b_hbm, p_hbm, out_hbm, *scratch):
    bufs = (scratch[0:4], scratch[4:8])
    sin = scratch[8:10]
    sout = scratch[10:12]
    per_w = n_sc // NW
    wid = lax.axis_index("s") * NC + lax.axis_index("c")
    base0 = wid * per_w
    nch = per_w // CH

    def in_copies(c, bset, sem):
        sl = pl.ds(base + base0 + c * CH, CH)
        q_v, a_v, b_v, p_v = bset
        srcs = (q_hbm.at[:, sl], a_hbm.at[sl], b_hbm.at[sl], p_hbm.at[sl])
        dsts = (q_v, a_v, b_v, p_v)
        return [pltpu.make_async_copy(s, d, sem) for s, d in zip(srcs, dsts)]

    def out_copies(c, bset, sem):
        sl = pl.ds(base0 + c * CH, CH)
        q_v = bset[0]
        return [pltpu.make_async_copy(q_v, out_hbm.at[:, sl], sem)]

    def compute(bset):
        q_v, a_v, b_v, p_v = bset
        lane = lax.iota(jnp.int32, LANES)

        @plsc.parallel_loop(0, CH // LANES, unroll=4)
        def body(i):
            s = pl.ds(i * LANES, LANES)
            av = a_v[s]
            bv = b_v[s]
            pv = p_v[s]
            pos = i * LANES + lane
            b2 = bv + bv
            pa = av + av + bv
            g0 = plsc.load_gather(q_v, [b2, pos])
            g1 = plsc.load_gather(q_v, [b2 + 1, pos])
            old = plsc.load_gather(q_v, [pa, pos])
            m = jnp.maximum(g0, g1)
            u = old + ALPHA * (pv + GAMMA * m - old)
            plsc.store_scatter(q_v, [pa, pos], u)

    for cp in in_copies(0, bufs[0], sin[0]):
        cp.start()

    def outer(t, carry):
        for b in (0, 1):
            cur = 2 * t + b
            nxt = cur + 1
            ob = 1 - b

            @pl.when(nxt < nch)
            def _prefetch():
                @pl.when(nxt >= 2)
                def _drain():
                    for cp in out_copies(nxt - 2, bufs[ob], sout[ob]):
                        cp.wait()

                for cp in in_copies(nxt, bufs[ob], sin[ob]):
                    cp.start()

            for cp in in_copies(cur, bufs[b], sin[b]):
                cp.wait()
            compute(bufs[b])
            for cp in out_copies(cur, bufs[b], sout[b]):
                cp.start()
        return carry

    lax.fori_loop(0, nch // 2, outer, 0)
    for cp in out_copies(nch - 2, bufs[0], sout[0]):
        cp.wait()
    for cp in out_copies(nch - 1, bufs[1], sout[1]):
        cp.wait()


@functools.lru_cache(maxsize=None)
def _make_sc_update(n_sc, base):
    mesh = plsc.VectorSubcoreMesh(
        core_axis_name="c", subcore_axis_name="s",
        num_cores=NC, num_subcores=NS,
    )
    return pl.kernel(
        functools.partial(_sc_body, n_sc, base),
        out_type=jax.ShapeDtypeStruct((4, n_sc), jnp.float32),
        mesh=mesh,
        compiler_params=pltpu.CompilerParams(needs_layout_passes=False),
        scratch_types=(
            [pltpu.VMEM((4, CH), jnp.float32)]
            + [pltpu.VMEM((CH,), jnp.int32)] * 2
            + [pltpu.VMEM((CH,), jnp.float32)]
        ) * 2 + [pltpu.SemaphoreType.DMA] * 4,
    )


def _tc_kernel(q_ref, a_ref, b_ref, p_ref, o_ref):
    q00 = q_ref[0, 0]
    q01 = q_ref[0, 1]
    q10 = q_ref[1, 0]
    q11 = q_ref[1, 1]
    a = a_ref[...]
    b = b_ref[...]
    p = p_ref[...]

    b0 = b == 0
    m = jnp.where(b0, jnp.maximum(q00, q01), jnp.maximum(q10, q11))
    old = jnp.where(
        a == 0, jnp.where(b0, q00, q01), jnp.where(b0, q10, q11)
    )
    u = old + ALPHA * (p + GAMMA * m - old)

    a0 = a == 0
    o_ref[0, 0] = jnp.where(a0 & b0, u, q00)
    o_ref[0, 1] = jnp.where(a0 & ~b0, u, q01)
    o_ref[1, 0] = jnp.where(~a0 & b0, u, q10)
    o_ref[1, 1] = jnp.where(~a0 & ~b0, u, q11)


def _tc_call(qt, a, b, p, n, n_tc):
    q_spec = pl.BlockSpec((2, 2, BLK), lambda i: (0, 0, i))
    v_spec = pl.BlockSpec((BLK,), lambda i: (i,))
    return pl.pallas_call(
        _tc_kernel,
        grid=(n_tc // BLK,),
        in_specs=[q_spec, v_spec, v_spec, v_spec],
        out_specs=q_spec,
        out_shape=jax.ShapeDtypeStruct((2, 2, n), jnp.float32),
        compiler_params=pltpu.CompilerParams(
            dimension_semantics=("arbitrary",),
        ),
    )(qt, a, b, p)


@jax.jit
def kernel(type_t_matrix, type_t1_matrix, Q_tensor, profit_matrix):
    n = type_t_matrix.size
    a = type_t_matrix.reshape(n).astype(jnp.int32)
    b = type_t1_matrix.reshape(n).astype(jnp.int32)
    p = profit_matrix.reshape(n).astype(jnp.float32)
    qt = jnp.transpose(Q_tensor, (1, 2, 0))
    q4 = qt.reshape(4, n)

    n_sc = (n // 8) // SC_GRAIN * SC_GRAIN
    if n_sc == 0 or (n - n_sc) % BLK:
        n_sc = n if n % SC_GRAIN == 0 else 0
    n_tc = n - n_sc

    if n_sc:
        sc_out = _make_sc_update(n_sc, n_tc)(q4, a, b, p)
    if n_tc:
        out = _tc_call(qt, a, b, p, n, n_tc)
        if n_sc:
            out = lax.dynamic_update_slice(
                out, sc_out.reshape(2, 2, n_sc), (0, 0, n_tc)
            )
    else:
        out = sc_out.reshape(2, 2, n)
    return jnp.transpose(out, (2, 0, 1))
